# Initial kernel scaffold; baseline (speedup 1.0000x reference)
import jax
import jax.numpy as jnp
from jax.experimental import pallas as pl


def kernel(sims, cc_ids, cc_embeds, cc_embed_mask, anchor_patches, anchor_embeds, anchor_mask, anchors_sim_index, W, b, Wp, bp):
    raise NotImplementedError("write your pallas kernel here")


# speedup vs baseline: 8.0803x; 8.0803x over previous
import functools

import jax
import jax.numpy as jnp
from jax.experimental import pallas as pl
from jax.experimental.pallas import tpu as pltpu

BATCH, MAX_N_CC, N_ANCHORS, D, NPO = 16, 64, 64, 256, 128
BC = BATCH * MAX_N_CC
G = 16


def _tc_body(col_ref, sims_ref, cc_ref, mask_ref, anchor_ref, W_ref, b_ref,
             wp_ref, bp_ref, out1_ref, out2_ref):
    col = col_ref[0]
    sims_blk = sims_ref[...]
    onehot = (jax.lax.broadcasted_iota(jnp.int32, (1, NPO), 1) == col)
    s = jnp.sum(jnp.where(onehot, sims_blk, 0.0), axis=1, keepdims=True)

    a = anchor_ref[...]
    m = mask_ref[...]
    am = a * m[:, :, None]
    Av = jnp.sum(am, axis=1)
    wp = wp_ref[...]
    q = jnp.sum(am * wp[0][None, None, :], axis=2)
    out2_ref[...] = jnp.maximum(s * q + bp_ref[0, 0], 0.0)

    aggr = s * Av
    dot = functools.partial(jax.lax.dot_general,
                            dimension_numbers=(((1,), (0,)), ((), ())),
                            precision=jax.lax.Precision.HIGHEST,
                            preferred_element_type=jnp.float32)
    out1_ref[...] = dot(aggr, W_ref[0:D, :]) + dot(cc_ref[...], W_ref[D:, :]) \
        + b_ref[...]


def kernel(sims, cc_ids, cc_embeds, cc_embed_mask, anchor_patches,
           anchor_embeds, anchor_mask, anchors_sim_index, W, b, Wp, bp):
    del cc_ids, cc_embed_mask, anchor_patches
    sims2 = sims.reshape(BC, NPO)
    cc2 = cc_embeds.reshape(BC, D)
    anchor3 = anchor_embeds.reshape(BC, N_ANCHORS, D)
    mask2 = anchor_mask.reshape(BC, N_ANCHORS).astype(jnp.float32)
    wp2 = Wp.reshape(1, D)
    b2 = b.reshape(1, D)
    bp2 = bp.reshape(1, 1).astype(jnp.float32)
    col = jnp.clip(jnp.asarray(anchors_sim_index, jnp.int32) * BC, 0, NPO - 1)
    col1 = col.reshape(1)

    grid = (BC // G,)
    out1, out2 = pl.pallas_call(
        _tc_body,
        grid_spec=pltpu.PrefetchScalarGridSpec(
            num_scalar_prefetch=1,
            grid=grid,
            in_specs=[
                pl.BlockSpec((G, NPO), lambda i, c: (i, 0)),
                pl.BlockSpec((G, D), lambda i, c: (i, 0)),
                pl.BlockSpec((G, N_ANCHORS), lambda i, c: (i, 0)),
                pl.BlockSpec((G, N_ANCHORS, D), lambda i, c: (i, 0, 0)),
                pl.BlockSpec((2 * D, D), lambda i, c: (0, 0)),
                pl.BlockSpec((1, D), lambda i, c: (0, 0)),
                pl.BlockSpec((1, D), lambda i, c: (0, 0)),
                pl.BlockSpec((1, 1), lambda i, c: (0, 0)),
            ],
            out_specs=[
                pl.BlockSpec((G, D), lambda i, c: (i, 0)),
                pl.BlockSpec((G, N_ANCHORS), lambda i, c: (i, 0)),
            ],
        ),
        out_shape=[
            jax.ShapeDtypeStruct((BC, D), jnp.float32),
            jax.ShapeDtypeStruct((BC, N_ANCHORS), jnp.float32),
        ],
        compiler_params=pltpu.CompilerParams(
            dimension_semantics=("parallel",),
        ),
    )(col1, sims2, cc2, mask2, anchor3, W, b2, wp2, bp2)

    return (out1.reshape(BATCH, MAX_N_CC, D),
            out2.reshape(BATCH, MAX_N_CC, N_ANCHORS))
